# Initial kernel scaffold; baseline (speedup 1.0000x reference)
import jax
import jax.numpy as jnp
from jax.experimental import pallas as pl


def kernel(x, edge_attr, edge_index, atom_tables, bond_tables, pre_W, pre_b, post_W, post_b):
    raise NotImplementedError("write your pallas kernel here")


# speedup vs baseline: 1.6801x; 1.6801x over previous
import functools

import jax
import jax.numpy as jnp
import numpy as np
from jax import lax
from jax.experimental import pallas as pl
from jax.experimental.pallas import tpu as pltpu

ATOM_DIMS = [119, 4, 12, 12, 10, 6, 6, 2, 2]
BOND_DIMS = [5, 6, 2]
H = 128
N_NODES = 10000
N_PAD = 10240
E = 320000
EPS = 1e-5
AVG_D_LOG = 1.0

ATOM_OFF = [0] + list(np.cumsum(ATOM_DIMS)[:-1])
BOND_OFF = [0] + list(np.cumsum(BOND_DIMS)[:-1])
ATOT = int(np.sum(ATOM_DIMS))
ATOT_PAD = 176
BTOT = int(np.sum(BOND_DIMS))
BTOT_PAD = 16

NB_NODE = 512
GN = N_PAD // NB_NODE
NB_EDGE = 8000
GE = E // NB_EDGE


def _encode_h_body(x_ref, atab_ref, h_ref):
    xb = x_ref[...]
    oh = jnp.zeros((NB_NODE, ATOT_PAD), jnp.float32)
    cols = lax.broadcasted_iota(jnp.int32, (NB_NODE, ATOT_PAD), 1)
    for f in range(len(ATOM_DIMS)):
        idx = xb[:, f:f + 1] + ATOM_OFF[f]
        oh = oh + (cols == idx).astype(jnp.float32)
    h_ref[...] = jnp.dot(oh, atab_ref[...], preferred_element_type=jnp.float32)


def _encode_h(x_pad, atab_pad):
    return pl.pallas_call(
        _encode_h_body,
        grid=(GN,),
        in_specs=[
            pl.BlockSpec((NB_NODE, len(ATOM_DIMS)), lambda i: (i, 0)),
            pl.BlockSpec((ATOT_PAD, H), lambda i: (0, 0)),
        ],
        out_specs=pl.BlockSpec((NB_NODE, H), lambda i: (i, 0)),
        out_shape=jax.ShapeDtypeStruct((N_PAD, H), jnp.float32),
    )(x_pad, atab_pad)


def _ab_body(h_ref, w1_ref, w2_ref, a_ref, b_ref):
    hb = h_ref[...]
    a_ref[...] = jnp.dot(hb, w1_ref[...], preferred_element_type=jnp.float32)
    b_ref[...] = jnp.dot(hb, w2_ref[...], preferred_element_type=jnp.float32)


def _ab(h, w1, w2):
    return pl.pallas_call(
        _ab_body,
        grid=(GN,),
        in_specs=[
            pl.BlockSpec((NB_NODE, H), lambda i: (i, 0)),
            pl.BlockSpec((H, H), lambda i: (0, 0)),
            pl.BlockSpec((H, H), lambda i: (0, 0)),
        ],
        out_specs=[
            pl.BlockSpec((NB_NODE, H), lambda i: (i, 0)),
            pl.BlockSpec((NB_NODE, H), lambda i: (i, 0)),
        ],
        out_shape=[
            jax.ShapeDtypeStruct((N_PAD, H), jnp.float32),
            jax.ShapeDtypeStruct((N_PAD, H), jnp.float32),
        ],
    )(h, w1, w2)


def _cb_body(ea_ref, btab_ref, w3_ref, preb_ref, cb_ref):
    eab = ea_ref[...]
    oh = jnp.zeros((NB_EDGE, BTOT_PAD), jnp.float32)
    cols = lax.broadcasted_iota(jnp.int32, (NB_EDGE, BTOT_PAD), 1)
    for f in range(len(BOND_DIMS)):
        idx = eab[:, f:f + 1] + BOND_OFF[f]
        oh = oh + (cols == idx).astype(jnp.float32)
    t3 = jnp.dot(btab_ref[...], w3_ref[...], preferred_element_type=jnp.float32)
    cb_ref[...] = jnp.dot(oh, t3, preferred_element_type=jnp.float32) + preb_ref[...]


def _cb(edge_attr, btab_pad, w3, preb):
    return pl.pallas_call(
        _cb_body,
        grid=(GE,),
        in_specs=[
            pl.BlockSpec((NB_EDGE, len(BOND_DIMS)), lambda i: (i, 0)),
            pl.BlockSpec((BTOT_PAD, H), lambda i: (0, 0)),
            pl.BlockSpec((H, H), lambda i: (0, 0)),
            pl.BlockSpec((1, H), lambda i: (0, 0)),
        ],
        out_specs=pl.BlockSpec((NB_EDGE, H), lambda i: (i, 0)),
        out_shape=jax.ShapeDtypeStruct((E, H), jnp.float32),
    )(edge_attr, btab_pad, w3, preb)


def _post_body(h_ref, s_ref, q_ref, mx_ref, mn_ref, bmat_ref, deg_ref,
               wp_ref, pb_ref, out_ref):
    hb = h_ref[...]
    bm = bmat_ref[...]
    su = s_ref[...]
    n = deg_ref[:, 0:1]
    nsafe = jnp.maximum(n, 1.0)
    pos = n > 0.0
    s = su + n * bm
    mean = s / nsafe
    msq = (q_ref[...] + 2.0 * bm * su + n * bm * bm) / nsafe
    mx = jnp.where(pos, mx_ref[...] + bm, 0.0)
    mn = jnp.where(pos, mn_ref[...] + bm, 0.0)
    var = jnp.maximum(msq - mean * mean, 0.0)
    std = jnp.sqrt(var + EPS)
    logd = jnp.log(n + 1.0)
    ampf = logd / AVG_D_LOG
    attf = AVG_D_LOG / jnp.where(logd > 0.0, logd, 1.0)
    wp = wp_ref[...]
    acc = jnp.dot(hb, wp[0:H], preferred_element_type=jnp.float32)
    aggs = (mean, mx, mn, std)
    for k in range(4):
        acc = acc + jnp.dot(aggs[k], wp[(1 + k) * H:(2 + k) * H],
                            preferred_element_type=jnp.float32)
    for k in range(4):
        acc = acc + jnp.dot(aggs[k] * ampf, wp[(5 + k) * H:(6 + k) * H],
                            preferred_element_type=jnp.float32)
    for k in range(4):
        acc = acc + jnp.dot(aggs[k] * attf, wp[(9 + k) * H:(10 + k) * H],
                            preferred_element_type=jnp.float32)
    out_ref[...] = acc + pb_ref[...] + hb


def _post(h, su, q, mx, mn, bmat, deg16, wp, pb):
    nspec = pl.BlockSpec((NB_NODE, H), lambda i: (i, 0))
    return pl.pallas_call(
        _post_body,
        grid=(GN,),
        in_specs=[
            nspec, nspec, nspec, nspec, nspec, nspec,
            pl.BlockSpec((NB_NODE, 16), lambda i: (i, 0)),
            pl.BlockSpec((13 * H, H), lambda i: (0, 0)),
            pl.BlockSpec((1, H), lambda i: (0, 0)),
        ],
        out_specs=nspec,
        out_shape=jax.ShapeDtypeStruct((N_PAD, H), jnp.float32),
    )(h, su, q, mx, mn, bmat, deg16, wp, pb)


def _edge_stage_jax(a_mat, cb, src, dst):
    u = a_mat[src] + cb
    su = jax.ops.segment_sum(u, dst, num_segments=N_PAD)
    q = jax.ops.segment_sum(u * u, dst, num_segments=N_PAD)
    mx = jax.ops.segment_max(u, dst, num_segments=N_PAD)
    mn = -jax.ops.segment_max(-u, dst, num_segments=N_PAD)
    deg = jax.ops.segment_sum(jnp.ones((E,), jnp.float32), dst,
                              num_segments=N_PAD)
    mx = jnp.where(deg[:, None] > 0, mx, 0.0)
    mn = jnp.where(deg[:, None] > 0, mn, 0.0)
    deg16 = jnp.broadcast_to(deg[:, None], (N_PAD, 16))
    return su, q, mx, mn, deg16


def kernel(x, edge_attr, edge_index, atom_tables, bond_tables,
           pre_W, pre_b, post_W, post_b):
    x = x.astype(jnp.int32)
    edge_attr = edge_attr.astype(jnp.int32)
    src = edge_index[0].astype(jnp.int32)
    dst = edge_index[1].astype(jnp.int32)
    x_pad = jnp.pad(x, ((0, N_PAD - N_NODES), (0, 0)))
    atab_pad = jnp.pad(atom_tables.astype(jnp.float32),
                       ((0, ATOT_PAD - ATOT), (0, 0)))
    btab_pad = jnp.pad(bond_tables.astype(jnp.float32),
                       ((0, BTOT_PAD - BTOT), (0, 0)))

    h = _encode_h(x_pad, atab_pad)

    for l in range(2):
        w1 = pre_W[l, 0:H]
        w2 = pre_W[l, H:2 * H]
        w3 = pre_W[l, 2 * H:3 * H]
        preb = pre_b[l][None, :]
        a_mat, b_mat = _ab(h, w1, w2)
        cb = _cb(edge_attr, btab_pad, w3, preb)
        su, q, mx, mn, deg16 = _edge_stage_jax(a_mat, cb, src, dst)
        h = _post(h, su, q, mx, mn, b_mat, deg16, post_W[l], post_b[l][None, :])

    return h[:N_NODES]
